# baseline (speedup 1.0000x reference)
import functools

import jax
import jax.numpy as jnp
from jax import lax
from jax.experimental import pallas as pl
from jax.experimental.pallas import tpu as pltpu
from jax.experimental.pallas import tpu_sc as plsc

_ACTION_WEIGHT = 10.0
_ACTION_DIM = 8
_OBS_DIM = 128
_TARGET_WEIGHT = 3.0

_NC = 2
_NS = 16
_CHUNK = 64


def _entities_block(pm, bbatch, ne, od):
    ad = _ACTION_DIM
    f32 = jnp.float32
    c = pm[:, :ne * od].reshape(bbatch * ne, od)
    main = c[:, ad:]
    wrapsrc = jnp.concatenate(
        [c[1:, :ad], jnp.zeros((1, ad), f32)], axis=0)
    tails = pm[:, ne * od:ne * od + ad]
    trep = jnp.broadcast_to(tails[:, None, :],
                            (bbatch, ne, ad)).reshape(bbatch * ne, ad)
    rows = jax.lax.broadcasted_iota(jnp.int32, (bbatch * ne, 1), 0)
    islast = (rows % ne) == (ne - 1)
    wrap = jnp.where(islast, trep, wrapsrc)
    return jnp.concatenate([main, wrap], axis=1)


def _match_body(horizon, bbatch, ne, od, row0, px_ref, py_ref,
                xal_ref, yal_ref, gi1_ref, gi2_ref, out_ref, asum_ref):
    g = pl.program_id(0)
    nsteps = pl.num_programs(0)
    f32 = jnp.float32

    iif = jax.lax.broadcasted_iota(jnp.int32, (ne, ne), 0).astype(f32)
    onesm = jnp.ones((ne, od), f32)

    @pl.when(g == 0)
    def _init():
        asum_ref[0] = f32(0.0)
        asum_ref[1] = f32(0.0)

    px = px_ref[:]
    py = py_ref[:]
    ex = _entities_block(px, bbatch, ne, od)
    ey = _entities_block(py, bbatch, ne, od)
    xal_ref[:] = ex
    yal_ref[:] = ey

    i1rows = []
    i2rows = []
    for r in range(bbatch):
        x = ex[r * ne:(r + 1) * ne]
        y = ey[r * ne:(r + 1) * ne]
        xa = jnp.concatenate([x * x, x], axis=1)
        ya = jnp.concatenate([y * y, y], axis=1)
        xb = jnp.concatenate([onesm, -2.0 * x], axis=1)
        yb = jnp.concatenate([onesm, -2.0 * y], axis=1)

        d1 = jax.lax.dot_general(xa, yb, (((1,), (1,)), ((), ())),
                                 preferred_element_type=f32)
        d2t = jax.lax.dot_general(ya, xb, (((1,), (1,)), ((), ())),
                                  preferred_element_type=f32)

        m1 = jnp.min(d1, axis=0, keepdims=True)
        idx1 = jnp.min(jnp.where(d1 <= m1, iif, f32(ne)), axis=0,
                       keepdims=True)
        m2t = jnp.min(d2t, axis=0, keepdims=True)
        idx2 = jnp.min(jnp.where(d2t <= m2t, iif, f32(ne)), axis=0,
                       keepdims=True)
        i1rows.append(idx1)
        i2rows.append(idx2)

    rowbase = (jax.lax.broadcasted_iota(jnp.int32, (bbatch, 1), 0)
               + g * bbatch).astype(f32) * f32(ne)
    gi1_ref[:] = (jnp.concatenate(i1rows, axis=0) + rowbase).astype(jnp.int32)
    gi2_ref[:] = (jnp.concatenate(i2rows, axis=0) + rowbase).astype(jnp.int32)

    ax = px[:, :_ACTION_DIM]
    ay = py[:, :_ACTION_DIM]
    al = jnp.sum(jnp.abs(ax - ay), axis=1, keepdims=True) / f32(_ACTION_DIM)
    kvec = (jax.lax.broadcasted_iota(jnp.int32, (bbatch, 1), 0)
            + g * bbatch + row0)
    is1 = jnp.mod(kvec, horizon) == 1
    w = jnp.where(is1, f32(_ACTION_WEIGHT), f32(1.0))
    asum_ref[0] += jnp.sum(al * w)
    asum_ref[1] += jnp.sum(jnp.where(is1, al, f32(0.0)))

    @pl.when(g == nsteps - 1)
    def _flush():
        li = jax.lax.broadcasted_iota(jnp.int32, (1, 128), 1)
        out_ref[0] = (jnp.where(li == 0, asum_ref[0], f32(0.0))
                      + jnp.where(li == 1, asum_ref[1], f32(0.0)))


def _sc_stage(xal, yal, gi1, gi2):
    nrows, od = xal.shape
    f32 = jnp.float32
    nw = _NC * _NS
    rows_per_w = nrows // nw
    nchunks = rows_per_w // _CHUNK
    nv = od // 16

    mesh = plsc.VectorSubcoreMesh(core_axis_name="c", subcore_axis_name="s")

    @functools.partial(
        pl.kernel, mesh=mesh,
        out_type=jax.ShapeDtypeStruct((nw, 2, 16), f32),
        scratch_types=(
            [pltpu.VMEM((rows_per_w,), jnp.int32)] * 2
            + [pltpu.VMEM((_CHUNK, od), f32)] * 8
            + [pltpu.VMEM((2, 16), f32)]
            + [pltpu.SemaphoreType.DMA] * 8
        ),
    )
    def sck(xal_h, yal_h, g1_h, g2_h, out_h,
            i1all, i2all,
            g1b0, g1b1, l1b0, l1b1, g2b0, g2b1, l2b0, l2b1,
            accv,
            sg10, sg11, sl10, sl11, sg20, sg21, sl20, sl21):
        wid = lax.axis_index("s") * _NC + lax.axis_index("c")
        base = wid * rows_per_w
        g1b = (g1b0, g1b1)
        l1b = (l1b0, l1b1)
        g2b = (g2b0, g2b1)
        l2b = (l2b0, l2b1)
        sg1 = (sg10, sg11)
        sl1 = (sl10, sl11)
        sg2 = (sg20, sg21)
        sl2 = (sl20, sl21)

        pltpu.sync_copy(g1_h.at[pl.ds(base, rows_per_w)], i1all)
        pltpu.sync_copy(g2_h.at[pl.ds(base, rows_per_w)], i2all)

        def issue(c, s):
            lb = c * _CHUNK
            cb = base + lb
            pltpu.async_copy(xal_h.at[i1all.at[pl.ds(lb, _CHUNK)]],
                             g1b[s], sg1[s])
            pltpu.async_copy(yal_h.at[pl.ds(cb, _CHUNK)], l1b[s], sl1[s])
            pltpu.async_copy(yal_h.at[i2all.at[pl.ds(lb, _CHUNK)]],
                             g2b[s], sg2[s])
            pltpu.async_copy(xal_h.at[pl.ds(cb, _CHUNK)], l2b[s], sl2[s])

        def drain(dst, sem):
            pltpu.make_async_copy(xal_h.at[pl.ds(0, _CHUNK)], dst, sem).wait()

        def accum(gbuf, linv, acc):
            def row(r, accs):
                out = []
                for v in range(nv):
                    sl = pl.ds(v * 16, 16)
                    out.append(accs[v]
                               + jnp.abs(gbuf[r, sl] - linv[r, sl]))
                return tuple(out)

            zero = tuple(jnp.zeros((16,), f32) for _ in range(nv))
            accs = lax.fori_loop(0, _CHUNK, row, zero)
            tot = acc
            for v in range(nv):
                tot = tot + accs[v]
            return tot

        issue(0, 0)

        def pair(c2, carry):
            a1, a2 = carry
            for s in range(2):
                c = c2 * 2 + s

                @pl.when(c + 1 < nchunks)
                def _pre():
                    issue(c + 1, 1 - s)

                drain(g1b[s], sg1[s])
                drain(l1b[s], sl1[s])
                a1 = accum(g1b[s], l1b[s], a1)
                drain(g2b[s], sg2[s])
                drain(l2b[s], sl2[s])
                a2 = accum(g2b[s], l2b[s], a2)
            return (a1, a2)

        z16 = jnp.zeros((16,), f32)
        a1, a2 = lax.fori_loop(0, nchunks // 2, pair, (z16, z16))
        accv[0] = a1
        accv[1] = a2
        pltpu.sync_copy(accv, out_h.at[wid])

    out = sck(xal, yal, gi1, gi2)
    s1 = jnp.sum(out[:, 0, :])
    s2 = jnp.sum(out[:, 1, :])
    return s1, s2


def _match_stage(p2, t2, nbh, horizon, bbatch, ne, od, row0):
    td = p2.shape[1]
    off = row0 // bbatch
    body = functools.partial(_match_body, horizon, bbatch, ne, od, row0)
    return pl.pallas_call(
        body,
        grid=(nbh // bbatch,),
        in_specs=[
            pl.BlockSpec((bbatch, td), lambda g: (g + off, 0)),
            pl.BlockSpec((bbatch, td), lambda g: (g + off, 0)),
        ],
        out_specs=[
            pl.BlockSpec((bbatch * ne, od), lambda g: (g, 0)),
            pl.BlockSpec((bbatch * ne, od), lambda g: (g, 0)),
            pl.BlockSpec((bbatch, ne), lambda g: (g, 0)),
            pl.BlockSpec((bbatch, ne), lambda g: (g, 0)),
            pl.BlockSpec((1, 1, 128), lambda g: (0, 0, 0)),
        ],
        out_shape=[
            jax.ShapeDtypeStruct((nbh * ne, od), jnp.float32),
            jax.ShapeDtypeStruct((nbh * ne, od), jnp.float32),
            jax.ShapeDtypeStruct((nbh, ne), jnp.int32),
            jax.ShapeDtypeStruct((nbh, ne), jnp.int32),
            jax.ShapeDtypeStruct((1, 1, 128), jnp.float32),
        ],
        scratch_shapes=[
            pltpu.SMEM((2,), jnp.float32),
        ],
    )(p2, t2)


def kernel(preds, targ):
    bs, horizon, td = preds.shape
    nb = bs * horizon
    ne = (td - _ACTION_DIM) // _OBS_DIM
    od = _OBS_DIM

    p2 = preds.reshape(nb, td)
    t2 = targ.reshape(nb, td)

    nsplit = 4
    bbatch = min(32, nb // nsplit)
    nbh = nb // nsplit

    s1 = jnp.float32(0.0)
    s2 = jnp.float32(0.0)
    acts = jnp.float32(0.0)
    a0s = jnp.float32(0.0)
    for h in range(nsplit):
        xal, yal, gi1, gi2, aout = _match_stage(
            p2, t2, nbh, horizon, bbatch, ne, od, h * nbh)
        hs1, hs2 = _sc_stage(xal, yal, gi1.reshape(nbh * ne),
                             gi2.reshape(nbh * ne))
        s1 = s1 + hs1
        s2 = s2 + hs2
        acts = acts + aout[0, 0, 0]
        a0s = a0s + aout[0, 0, 1]

    csum = _TARGET_WEIGHT * s1 + s2

    chamfer_loss = csum / (_TARGET_WEIGHT + 1.0) / (nb * ne * od)
    action_loss = acts / nb
    a0_loss = a0s / bs
    loss = action_loss + chamfer_loss
    return (loss, a0_loss)


# speedup vs baseline: 1.0263x; 1.0263x over previous
import functools

import jax
import jax.numpy as jnp
from jax import lax
from jax.experimental import pallas as pl
from jax.experimental.pallas import tpu as pltpu
from jax.experimental.pallas import tpu_sc as plsc

_ACTION_WEIGHT = 10.0
_ACTION_DIM = 8
_OBS_DIM = 128
_TARGET_WEIGHT = 3.0

_NC = 2
_NS = 16
_CHUNK = 64


def _entities_block(pm, bbatch, ne, od):
    ad = _ACTION_DIM
    f32 = jnp.float32
    c = pm[:, :ne * od].reshape(bbatch * ne, od)
    main = c[:, ad:]
    wrapsrc = jnp.concatenate(
        [c[1:, :ad], jnp.zeros((1, ad), f32)], axis=0)
    tails = pm[:, ne * od:ne * od + ad]
    trep = jnp.broadcast_to(tails[:, None, :],
                            (bbatch, ne, ad)).reshape(bbatch * ne, ad)
    rows = jax.lax.broadcasted_iota(jnp.int32, (bbatch * ne, 1), 0)
    islast = (rows % ne) == (ne - 1)
    wrap = jnp.where(islast, trep, wrapsrc)
    return jnp.concatenate([main, wrap], axis=1)


def _match_body(horizon, bbatch, ne, od, row0, px_ref, py_ref,
                xal_ref, yal_ref, gi1_ref, gi2_ref, out_ref, asum_ref):
    g = pl.program_id(0)
    nsteps = pl.num_programs(0)
    f32 = jnp.float32

    iif = jax.lax.broadcasted_iota(jnp.int32, (ne, ne), 0).astype(f32)
    onesm = jnp.ones((ne, od), f32)

    @pl.when(g == 0)
    def _init():
        asum_ref[0] = f32(0.0)
        asum_ref[1] = f32(0.0)

    px = px_ref[:]
    py = py_ref[:]
    ex = _entities_block(px, bbatch, ne, od)
    ey = _entities_block(py, bbatch, ne, od)
    xal_ref[:] = ex
    yal_ref[:] = ey

    i1rows = []
    i2rows = []
    for r in range(bbatch):
        x = ex[r * ne:(r + 1) * ne]
        y = ey[r * ne:(r + 1) * ne]
        xa = jnp.concatenate([x * x, x], axis=1)
        ya = jnp.concatenate([y * y, y], axis=1)
        xb = jnp.concatenate([onesm, -2.0 * x], axis=1)
        yb = jnp.concatenate([onesm, -2.0 * y], axis=1)

        d1 = jax.lax.dot_general(xa, yb, (((1,), (1,)), ((), ())),
                                 preferred_element_type=f32)
        d2t = jax.lax.dot_general(ya, xb, (((1,), (1,)), ((), ())),
                                  preferred_element_type=f32)

        m1 = jnp.min(d1, axis=0, keepdims=True)
        idx1 = jnp.min(jnp.where(d1 <= m1, iif, f32(ne)), axis=0,
                       keepdims=True)
        m2t = jnp.min(d2t, axis=0, keepdims=True)
        idx2 = jnp.min(jnp.where(d2t <= m2t, iif, f32(ne)), axis=0,
                       keepdims=True)
        i1rows.append(idx1)
        i2rows.append(idx2)

    rowbase = (jax.lax.broadcasted_iota(jnp.int32, (bbatch, 1), 0)
               + g * bbatch).astype(f32) * f32(ne)
    gi1_ref[:] = (jnp.concatenate(i1rows, axis=0) + rowbase).astype(jnp.int32)
    gi2_ref[:] = (jnp.concatenate(i2rows, axis=0) + rowbase).astype(jnp.int32)

    ax = px[:, :_ACTION_DIM]
    ay = py[:, :_ACTION_DIM]
    al = jnp.sum(jnp.abs(ax - ay), axis=1, keepdims=True) / f32(_ACTION_DIM)
    kvec = (jax.lax.broadcasted_iota(jnp.int32, (bbatch, 1), 0)
            + g * bbatch + row0)
    is1 = jnp.mod(kvec, horizon) == 1
    w = jnp.where(is1, f32(_ACTION_WEIGHT), f32(1.0))
    asum_ref[0] += jnp.sum(al * w)
    asum_ref[1] += jnp.sum(jnp.where(is1, al, f32(0.0)))

    @pl.when(g == nsteps - 1)
    def _flush():
        li = jax.lax.broadcasted_iota(jnp.int32, (1, 128), 1)
        out_ref[0] = (jnp.where(li == 0, asum_ref[0], f32(0.0))
                      + jnp.where(li == 1, asum_ref[1], f32(0.0)))


def _sc_stage(xal, yal, gi1, gi2):
    nrows, odp = xal.shape
    f32 = jnp.float32
    bf16 = jnp.bfloat16
    nw = _NC * _NS
    rows_per_w = nrows // nw
    nchunks = rows_per_w // _CHUNK
    nv = odp // 16

    mesh = plsc.VectorSubcoreMesh(core_axis_name="c", subcore_axis_name="s")

    @functools.partial(
        pl.kernel, mesh=mesh,
        out_type=jax.ShapeDtypeStruct((nw, 2, 16), f32),
        scratch_types=(
            [pltpu.VMEM((rows_per_w,), jnp.int32)] * 2
            + [pltpu.VMEM((_CHUNK, odp), f32)] * 8
            + [pltpu.VMEM((2, 16), f32)]
            + [pltpu.SemaphoreType.DMA] * 8
        ),
    )
    def sck(xal_h, yal_h, g1_h, g2_h, out_h,
            i1all, i2all,
            g1b0, g1b1, l1b0, l1b1, g2b0, g2b1, l2b0, l2b1,
            accv,
            sg10, sg11, sl10, sl11, sg20, sg21, sl20, sl21):
        wid = lax.axis_index("s") * _NC + lax.axis_index("c")
        base = wid * rows_per_w
        g1b = (g1b0, g1b1)
        l1b = (l1b0, l1b1)
        g2b = (g2b0, g2b1)
        l2b = (l2b0, l2b1)
        sg1 = (sg10, sg11)
        sl1 = (sl10, sl11)
        sg2 = (sg20, sg21)
        sl2 = (sl20, sl21)

        pltpu.sync_copy(g1_h.at[pl.ds(base, rows_per_w)], i1all)
        pltpu.sync_copy(g2_h.at[pl.ds(base, rows_per_w)], i2all)

        def issue(c, s):
            lb = c * _CHUNK
            cb = base + lb
            pltpu.async_copy(xal_h.at[i1all.at[pl.ds(lb, _CHUNK)]],
                             g1b[s], sg1[s])
            pltpu.async_copy(yal_h.at[pl.ds(cb, _CHUNK)], l1b[s], sl1[s])
            pltpu.async_copy(yal_h.at[i2all.at[pl.ds(lb, _CHUNK)]],
                             g2b[s], sg2[s])
            pltpu.async_copy(xal_h.at[pl.ds(cb, _CHUNK)], l2b[s], sl2[s])

        def drain(dst, sem):
            pltpu.make_async_copy(xal_h.at[pl.ds(0, _CHUNK)], dst, sem).wait()

        def accum(gbuf, linv, acc):
            def row(r, accs):
                out = []
                for v in range(nv):
                    sl = pl.ds(v * 16, 16)
                    out.append(accs[v] + jnp.abs(gbuf[r, sl] - linv[r, sl]))
                return tuple(out)

            zero = tuple(jnp.zeros((16,), f32) for _ in range(nv))
            accs = lax.fori_loop(0, _CHUNK, row, zero)
            tot = acc
            for v in range(nv):
                tot = tot + accs[v]
            return tot

        issue(0, 0)

        def pair(c2, carry):
            a1, a2 = carry
            for s in range(2):
                c = c2 * 2 + s

                @pl.when(c + 1 < nchunks)
                def _pre():
                    issue(c + 1, 1 - s)

                drain(g1b[s], sg1[s])
                drain(l1b[s], sl1[s])
                a1 = accum(g1b[s], l1b[s], a1)
                drain(g2b[s], sg2[s])
                drain(l2b[s], sl2[s])
                a2 = accum(g2b[s], l2b[s], a2)
            return (a1, a2)

        z16 = jnp.zeros((16,), f32)
        a1, a2 = lax.fori_loop(0, nchunks // 2, pair, (z16, z16))
        accv[0] = a1
        accv[1] = a2
        pltpu.sync_copy(accv, out_h.at[wid])

    out = sck(xal, yal, gi1, gi2)
    s1 = jnp.sum(out[:, 0, :])
    s2 = jnp.sum(out[:, 1, :])
    return s1, s2


def _match_stage(p2, t2, nbh, horizon, bbatch, ne, od, row0):
    td = p2.shape[1]
    off = row0 // bbatch
    body = functools.partial(_match_body, horizon, bbatch, ne, od, row0)
    return pl.pallas_call(
        body,
        grid=(nbh // bbatch,),
        in_specs=[
            pl.BlockSpec((bbatch, td), lambda g: (g + off, 0)),
            pl.BlockSpec((bbatch, td), lambda g: (g + off, 0)),
        ],
        out_specs=[
            pl.BlockSpec((bbatch * ne, od), lambda g: (g, 0)),
            pl.BlockSpec((bbatch * ne, od), lambda g: (g, 0)),
            pl.BlockSpec((bbatch, ne), lambda g: (g, 0)),
            pl.BlockSpec((bbatch, ne), lambda g: (g, 0)),
            pl.BlockSpec((1, 1, 128), lambda g: (0, 0, 0)),
        ],
        out_shape=[
            jax.ShapeDtypeStruct((nbh * ne, od), jnp.float32),
            jax.ShapeDtypeStruct((nbh * ne, od), jnp.float32),
            jax.ShapeDtypeStruct((nbh, ne), jnp.int32),
            jax.ShapeDtypeStruct((nbh, ne), jnp.int32),
            jax.ShapeDtypeStruct((1, 1, 128), jnp.float32),
        ],
        scratch_shapes=[
            pltpu.SMEM((2,), jnp.float32),
        ],
    )(p2, t2)


def kernel(preds, targ):
    bs, horizon, td = preds.shape
    nb = bs * horizon
    ne = (td - _ACTION_DIM) // _OBS_DIM
    od = _OBS_DIM

    p2 = preds.reshape(nb, td)
    t2 = targ.reshape(nb, td)

    nsplit = 2
    bbatch = min(32, nb // nsplit)
    nbh = nb // nsplit

    s1 = jnp.float32(0.0)
    s2 = jnp.float32(0.0)
    acts = jnp.float32(0.0)
    a0s = jnp.float32(0.0)
    for h in range(nsplit):
        xal, yal, gi1, gi2, aout = _match_stage(
            p2, t2, nbh, horizon, bbatch, ne, od, h * nbh)
        hs1, hs2 = _sc_stage(xal, yal, gi1.reshape(nbh * ne),
                             gi2.reshape(nbh * ne))
        s1 = s1 + hs1
        s2 = s2 + hs2
        acts = acts + aout[0, 0, 0]
        a0s = a0s + aout[0, 0, 1]

    csum = _TARGET_WEIGHT * s1 + s2

    chamfer_loss = csum / (_TARGET_WEIGHT + 1.0) / (nb * ne * od)
    action_loss = acts / nb
    a0_loss = a0s / bs
    loss = action_loss + chamfer_loss
    return (loss, a0_loss)


# speedup vs baseline: 1.0328x; 1.0063x over previous
import functools

import jax
import jax.numpy as jnp
from jax import lax
from jax.experimental import pallas as pl
from jax.experimental.pallas import tpu as pltpu
from jax.experimental.pallas import tpu_sc as plsc

_ACTION_WEIGHT = 10.0
_ACTION_DIM = 8
_OBS_DIM = 128
_TARGET_WEIGHT = 3.0

_NC = 2
_NS = 16
_CHUNK = 64


def _entities_block(pm, bbatch, ne, od):
    ad = _ACTION_DIM
    f32 = jnp.float32
    c = pm[:, :ne * od].reshape(bbatch * ne, od)
    main = c[:, ad:]
    wrapsrc = jnp.concatenate(
        [c[1:, :ad], jnp.zeros((1, ad), f32)], axis=0)
    tails = pm[:, ne * od:ne * od + ad]
    trep = jnp.broadcast_to(tails[:, None, :],
                            (bbatch, ne, ad)).reshape(bbatch * ne, ad)
    rows = jax.lax.broadcasted_iota(jnp.int32, (bbatch * ne, 1), 0)
    islast = (rows % ne) == (ne - 1)
    wrap = jnp.where(islast, trep, wrapsrc)
    return jnp.concatenate([main, wrap], axis=1)


def _match_body(horizon, bbatch, ne, od, row0, px_ref, py_ref,
                xal_ref, yal_ref, gi1_ref, gi2_ref, out_ref, asum_ref):
    g = pl.program_id(0)
    nsteps = pl.num_programs(0)
    f32 = jnp.float32

    iif = jax.lax.broadcasted_iota(jnp.int32, (ne, ne), 0).astype(f32)
    onesm = jnp.ones((ne, od), f32)

    @pl.when(g == 0)
    def _init():
        asum_ref[0] = f32(0.0)
        asum_ref[1] = f32(0.0)

    px = px_ref[:]
    py = py_ref[:]
    ex = _entities_block(px, bbatch, ne, od)
    ey = _entities_block(py, bbatch, ne, od)
    xal_ref[:] = ex
    yal_ref[:] = ey

    i1rows = []
    i2rows = []
    for r in range(bbatch):
        x = ex[r * ne:(r + 1) * ne]
        y = ey[r * ne:(r + 1) * ne]
        xa = jnp.concatenate([x * x, x], axis=1)
        ya = jnp.concatenate([y * y, y], axis=1)
        xb = jnp.concatenate([onesm, -2.0 * x], axis=1)
        yb = jnp.concatenate([onesm, -2.0 * y], axis=1)

        d1 = jax.lax.dot_general(xa, yb, (((1,), (1,)), ((), ())),
                                 preferred_element_type=f32)
        d2t = jax.lax.dot_general(ya, xb, (((1,), (1,)), ((), ())),
                                  preferred_element_type=f32)

        m1 = jnp.min(d1, axis=0, keepdims=True)
        idx1 = jnp.min(jnp.where(d1 <= m1, iif, f32(ne)), axis=0,
                       keepdims=True)
        m2t = jnp.min(d2t, axis=0, keepdims=True)
        idx2 = jnp.min(jnp.where(d2t <= m2t, iif, f32(ne)), axis=0,
                       keepdims=True)
        i1rows.append(idx1)
        i2rows.append(idx2)

    rowbase = (jax.lax.broadcasted_iota(jnp.int32, (bbatch, 1), 0)
               + g * bbatch).astype(f32) * f32(ne)
    gi1_ref[:] = (jnp.concatenate(i1rows, axis=0) + rowbase).astype(jnp.int32)
    gi2_ref[:] = (jnp.concatenate(i2rows, axis=0) + rowbase).astype(jnp.int32)

    ax = px[:, :_ACTION_DIM]
    ay = py[:, :_ACTION_DIM]
    al = jnp.sum(jnp.abs(ax - ay), axis=1, keepdims=True) / f32(_ACTION_DIM)
    kvec = (jax.lax.broadcasted_iota(jnp.int32, (bbatch, 1), 0)
            + g * bbatch + row0)
    is1 = jnp.mod(kvec, horizon) == 1
    w = jnp.where(is1, f32(_ACTION_WEIGHT), f32(1.0))
    asum_ref[0] += jnp.sum(al * w)
    asum_ref[1] += jnp.sum(jnp.where(is1, al, f32(0.0)))

    @pl.when(g == nsteps - 1)
    def _flush():
        li = jax.lax.broadcasted_iota(jnp.int32, (1, 128), 1)
        out_ref[0] = (jnp.where(li == 0, asum_ref[0], f32(0.0))
                      + jnp.where(li == 1, asum_ref[1], f32(0.0)))


def _sc_stage(xal, yal, gi1, gi2):
    nrows, odp = xal.shape
    f32 = jnp.float32
    nw = _NC * _NS
    rows_per_w = nrows // nw
    nchunks = rows_per_w // _CHUNK
    nv = odp // 16

    mesh = plsc.VectorSubcoreMesh(core_axis_name="c", subcore_axis_name="s")

    @functools.partial(
        pl.kernel, mesh=mesh,
        out_type=jax.ShapeDtypeStruct((nw, 2, 16), f32),
        scratch_types=(
            [pltpu.VMEM((rows_per_w,), jnp.int32)] * 2
            + [pltpu.VMEM((_CHUNK, odp), f32)] * 8
            + [pltpu.VMEM((2, 16), f32)]
            + [pltpu.SemaphoreType.DMA] * 8
        ),
    )
    def sck(xal_h, yal_h, g1_h, g2_h, out_h,
            i1all, i2all,
            g1b0, g1b1, l1b0, l1b1, g2b0, g2b1, l2b0, l2b1,
            accv,
            sg10, sg11, sl10, sl11, sg20, sg21, sl20, sl21):
        wid = lax.axis_index("s") * _NC + lax.axis_index("c")
        base = wid * rows_per_w
        g1b = (g1b0, g1b1)
        l1b = (l1b0, l1b1)
        g2b = (g2b0, g2b1)
        l2b = (l2b0, l2b1)
        sg1 = (sg10, sg11)
        sl1 = (sl10, sl11)
        sg2 = (sg20, sg21)
        sl2 = (sl20, sl21)

        pltpu.sync_copy(g1_h.at[pl.ds(base, rows_per_w)], i1all)
        pltpu.sync_copy(g2_h.at[pl.ds(base, rows_per_w)], i2all)

        def issue(c, s):
            lb = c * _CHUNK
            cb = base + lb
            pltpu.async_copy(xal_h.at[i1all.at[pl.ds(lb, _CHUNK)]],
                             g1b[s], sg1[s])
            pltpu.async_copy(yal_h.at[pl.ds(cb, _CHUNK)], l1b[s], sl1[s])
            pltpu.async_copy(yal_h.at[i2all.at[pl.ds(lb, _CHUNK)]],
                             g2b[s], sg2[s])
            pltpu.async_copy(xal_h.at[pl.ds(cb, _CHUNK)], l2b[s], sl2[s])

        def drain(dst, sem):
            pltpu.make_async_copy(xal_h.at[pl.ds(0, _CHUNK)], dst, sem).wait()

        def accum(gbuf, linv, acc):
            def row(r, accs):
                out = []
                for v in range(nv):
                    sl = pl.ds(v * 16, 16)
                    out.append(accs[v] + jnp.abs(gbuf[r, sl] - linv[r, sl]))
                return tuple(out)

            zero = tuple(jnp.zeros((16,), f32) for _ in range(nv))
            accs = lax.fori_loop(0, _CHUNK, row, zero)
            tot = acc
            for v in range(nv):
                tot = tot + accs[v]
            return tot

        issue(0, 0)

        def pair(c2, carry):
            a1, a2 = carry
            for s in range(2):
                c = c2 * 2 + s

                @pl.when(c + 1 < nchunks)
                def _pre():
                    issue(c + 1, 1 - s)

                drain(g1b[s], sg1[s])
                drain(l1b[s], sl1[s])
                a1 = accum(g1b[s], l1b[s], a1)
                drain(g2b[s], sg2[s])
                drain(l2b[s], sl2[s])
                a2 = accum(g2b[s], l2b[s], a2)
            return (a1, a2)

        z16 = jnp.zeros((16,), f32)
        a1, a2 = lax.fori_loop(0, nchunks // 2, pair, (z16, z16))
        accv[0] = a1
        accv[1] = a2
        pltpu.sync_copy(accv, out_h.at[wid])

    out = sck(xal, yal, gi1, gi2)
    s1 = jnp.sum(out[:, 0, :])
    s2 = jnp.sum(out[:, 1, :])
    return s1, s2


def _match_stage(p2, t2, nbh, horizon, bbatch, ne, od, row0):
    td = p2.shape[1]
    off = row0 // bbatch
    body = functools.partial(_match_body, horizon, bbatch, ne, od, row0)
    return pl.pallas_call(
        body,
        grid=(nbh // bbatch,),
        in_specs=[
            pl.BlockSpec((bbatch, td), lambda g: (g + off, 0)),
            pl.BlockSpec((bbatch, td), lambda g: (g + off, 0)),
        ],
        out_specs=[
            pl.BlockSpec((bbatch * ne, od), lambda g: (g, 0)),
            pl.BlockSpec((bbatch * ne, od), lambda g: (g, 0)),
            pl.BlockSpec((bbatch, ne), lambda g: (g, 0)),
            pl.BlockSpec((bbatch, ne), lambda g: (g, 0)),
            pl.BlockSpec((1, 1, 128), lambda g: (0, 0, 0)),
        ],
        out_shape=[
            jax.ShapeDtypeStruct((nbh * ne, od), jnp.float32),
            jax.ShapeDtypeStruct((nbh * ne, od), jnp.float32),
            jax.ShapeDtypeStruct((nbh, ne), jnp.int32),
            jax.ShapeDtypeStruct((nbh, ne), jnp.int32),
            jax.ShapeDtypeStruct((1, 1, 128), jnp.float32),
        ],
        scratch_shapes=[
            pltpu.SMEM((2,), jnp.float32),
        ],
    )(p2, t2)


def kernel(preds, targ):
    bs, horizon, td = preds.shape
    nb = bs * horizon
    ne = (td - _ACTION_DIM) // _OBS_DIM
    od = _OBS_DIM

    p2 = preds.reshape(nb, td)
    t2 = targ.reshape(nb, td)

    nsplit = 2
    bbatch = min(64, nb // nsplit)
    nbh = nb // nsplit

    s1 = jnp.float32(0.0)
    s2 = jnp.float32(0.0)
    acts = jnp.float32(0.0)
    a0s = jnp.float32(0.0)
    for h in range(nsplit):
        xal, yal, gi1, gi2, aout = _match_stage(
            p2, t2, nbh, horizon, bbatch, ne, od, h * nbh)
        hs1, hs2 = _sc_stage(xal, yal, gi1.reshape(nbh * ne),
                             gi2.reshape(nbh * ne))
        s1 = s1 + hs1
        s2 = s2 + hs2
        acts = acts + aout[0, 0, 0]
        a0s = a0s + aout[0, 0, 1]

    csum = _TARGET_WEIGHT * s1 + s2

    chamfer_loss = csum / (_TARGET_WEIGHT + 1.0) / (nb * ne * od)
    action_loss = acts / nb
    a0_loss = a0s / bs
    loss = action_loss + chamfer_loss
    return (loss, a0_loss)
